# baseline (speedup 1.0000x reference)
import functools

import jax
import jax.numpy as jnp
from jax import lax
from jax.experimental import pallas as pl
from jax.experimental.pallas import tpu as pltpu
from jax.experimental.pallas import tpu_sc as plsc

NUM_WORKERS = 32
CHUNK = 512


def _emb_body(idx_hbm, table_hbm, out_hbm, idx_v, rows_v, trans_v, g0, g1,
              wsem, *, per_w, nchunks, nb, d_dim):
    wid = lax.axis_index("s") * 2 + lax.axis_index("c")
    base_c = wid * nchunks
    pltpu.sync_copy(idx_hbm.at[pl.ds(base_c * CHUNK, per_w)], idx_v)
    gsem = (g0, g1)
    iota = lax.iota(jnp.int32, 16)

    def g_start(i, b):
        pltpu.make_async_copy(table_hbm.at[idx_v.at[pl.ds(i * CHUNK, CHUNK)]],
                              rows_v.at[b], gsem[b]).start()

    def g_wait(b):
        pltpu.make_async_copy(table_hbm.at[idx_v.at[pl.ds(0, CHUNK)]],
                              rows_v.at[b], gsem[b]).wait()

    qrows = [16 * q + iota for q in range(d_dim // 16)]

    def transpose(b):
        rows2 = rows_v.at[b]

        @plsc.parallel_loop(0, CHUNK, step=1, unroll=4)
        def tj(j):
            cj = jnp.full((16,), j, jnp.int32)
            for q in range(d_dim // 16):
                v = rows2[j, pl.ds(16 * q, 16)]
                plsc.store_scatter(trans_v, [qrows[q], cj], v)

    def w_start(i):
        c = base_c + i
        h = c // nb
        b0 = (c - h * nb) * CHUNK
        pltpu.make_async_copy(trans_v,
                              out_hbm.at[h, :, pl.ds(b0, CHUNK)], wsem).start()

    def w_wait():
        pltpu.make_async_copy(trans_v,
                              out_hbm.at[0, :, pl.ds(0, CHUNK)], wsem).wait()

    g_start(0, 0)
    g_start(1, 1)
    g_wait(0)
    transpose(0)
    w_start(0)

    def pair(go, carry):
        u = 2 * go + 1
        g_start(u + 1, 0)
        g_wait(1)
        w_wait()
        transpose(1)
        w_start(u)
        g_start(u + 2, 1)
        g_wait(0)
        w_wait()
        transpose(0)
        w_start(u + 1)
        return carry

    lax.fori_loop(0, (nchunks - 2) // 2, pair, 0)

    g_wait(1)
    w_wait()
    transpose(1)
    w_start(nchunks - 1)
    w_wait()


def kernel(token_id, table):
    B, H = token_id.shape
    V, D = table.shape
    N = B * H
    per_w = N // NUM_WORKERS
    nchunks = per_w // CHUNK
    nb = B // CHUNK
    idx = token_id.T.reshape(N).astype(jnp.int32)

    mesh = plsc.VectorSubcoreMesh(core_axis_name="c", subcore_axis_name="s")
    emb = functools.partial(
        pl.kernel,
        mesh=mesh,
        out_type=jax.ShapeDtypeStruct((H, D, B), jnp.float32),
        scratch_types=[
            pltpu.VMEM((per_w,), jnp.int32),
            pltpu.VMEM((2, CHUNK, D), jnp.float32),
            pltpu.VMEM((D, CHUNK), jnp.float32),
            pltpu.SemaphoreType.DMA,
            pltpu.SemaphoreType.DMA,
            pltpu.SemaphoreType.DMA,
        ],
        compiler_params=pltpu.CompilerParams(use_tc_tiling_on_sc=False,
                                             needs_layout_passes=False),
    )(functools.partial(_emb_body, per_w=per_w, nchunks=nchunks, nb=nb,
                        d_dim=D))

    outp = emb(idx, table)
    return jnp.transpose(outp, (2, 0, 1))


# speedup vs baseline: 1.2605x; 1.2605x over previous
import functools

import jax
import jax.numpy as jnp
from jax import lax
from jax.experimental import pallas as pl
from jax.experimental.pallas import tpu as pltpu
from jax.experimental.pallas import tpu_sc as plsc

NUM_WORKERS = 32
CHUNK = 512


def _emb_body(tid_hbm, table_hbm, out_hbm, idx_v, rows_v, g0, g1, w0, w1,
              *, h_dim):
    wid = lax.axis_index("s") * 2 + lax.axis_index("c")
    b0 = wid * CHUNK
    pltpu.sync_copy(tid_hbm.at[:, pl.ds(b0, CHUNK)], idx_v)
    gsem = (g0, g1)
    wsem = (w0, w1)

    def g_start(h, b):
        pltpu.make_async_copy(table_hbm.at[idx_v.at[h]], rows_v.at[b],
                              gsem[b]).start()

    def g_wait(b):
        pltpu.make_async_copy(table_hbm.at[idx_v.at[0]], rows_v.at[b],
                              gsem[b]).wait()

    def w_start(h, b):
        pltpu.make_async_copy(rows_v.at[b], out_hbm.at[pl.ds(b0, CHUNK), h, :],
                              wsem[b]).start()

    def w_wait(b):
        pltpu.make_async_copy(rows_v.at[b], out_hbm.at[pl.ds(0, CHUNK), 0, :],
                              wsem[b]).wait()

    g_start(0, 0)
    g_start(1, 1)
    g_wait(0)
    w_start(0, 0)

    def pair(go, carry):
        u = 2 * go + 1
        g_wait(1)
        w_start(u, 1)
        w_wait(0)
        g_start(u + 1, 0)
        g_wait(0)
        w_start(u + 1, 0)
        w_wait(1)
        g_start(u + 2, 1)
        return carry

    lax.fori_loop(0, (h_dim - 2) // 2, pair, 0)

    g_wait(1)
    w_start(h_dim - 1, 1)
    w_wait(0)
    w_wait(1)


def kernel(token_id, table):
    B, H = token_id.shape
    V, D = table.shape
    tid2 = token_id.T.astype(jnp.int32)

    mesh = plsc.VectorSubcoreMesh(core_axis_name="c", subcore_axis_name="s")
    emb = functools.partial(
        pl.kernel,
        mesh=mesh,
        out_type=jax.ShapeDtypeStruct((B, H, D), jnp.float32),
        scratch_types=[
            pltpu.VMEM((H, CHUNK), jnp.int32),
            pltpu.VMEM((2, CHUNK, D), jnp.float32),
            pltpu.SemaphoreType.DMA,
            pltpu.SemaphoreType.DMA,
            pltpu.SemaphoreType.DMA,
            pltpu.SemaphoreType.DMA,
        ],
        compiler_params=pltpu.CompilerParams(use_tc_tiling_on_sc=False,
                                             needs_layout_passes=False),
    )(functools.partial(_emb_body, h_dim=H))

    return emb(tid2, table)
